# Initial kernel scaffold; baseline (speedup 1.0000x reference)
import jax
import jax.numpy as jnp
from jax.experimental import pallas as pl


def kernel(input_ids, attention_mask, embedding_table, dense_kernel):
    raise NotImplementedError("write your pallas kernel here")


# speedup vs baseline: 16.0693x; 16.0693x over previous
import functools

import jax
import jax.numpy as jnp
from jax import lax
from jax.experimental import pallas as pl
from jax.experimental.pallas import tpu as pltpu
from jax.experimental.pallas import tpu_sc as plsc

_B = 4096
_L = 200
_D = 64
_NC = 2
_NS = 16
_NW = _NC * _NS
_ROWS_W = _B // _NW
_CHUNK = 100
_CPR = _L // _CHUNK
_NCHUNK = _ROWS_W * _CPR
_NBUF = 4


def _sumsq_sc(ids2, table):
    mesh = plsc.VectorSubcoreMesh(core_axis_name="c", subcore_axis_name="s")

    @functools.partial(
        pl.kernel,
        out_type=jax.ShapeDtypeStruct((_B, _D), jnp.float32),
        mesh=mesh,
        compiler_params=pltpu.CompilerParams(use_tc_tiling_on_sc=False),
        scratch_types=(
            [
                pltpu.VMEM((_NCHUNK, _CHUNK), jnp.int32),
                pltpu.VMEM((_ROWS_W, _D), jnp.float32),
            ]
            + [pltpu.VMEM((_CHUNK, _D), jnp.float32) for _ in range(_NBUF)]
            + [pltpu.SemaphoreType.DMA for _ in range(_NBUF)]
        ),
    )
    def k(ids_hbm, tab_hbm, out_hbm, ids_v, out_v, b0, b1, b2, b3, s0, s1, s2, s3):
        bufs = (b0, b1, b2, b3)
        sems = (s0, s1, s2, s3)
        wid = lax.axis_index("s") * _NC + lax.axis_index("c")
        pltpu.sync_copy(ids_hbm.at[pl.ds(wid * _NCHUNK, _NCHUNK)], ids_v)

        def start(c, b):
            pltpu.make_async_copy(tab_hbm.at[ids_v.at[c]], bufs[b], sems[b]).start()

        def wait(c, b):
            pltpu.make_async_copy(tab_hbm.at[ids_v.at[c]], bufs[b], sems[b]).wait()

        for b in range(_NBUF):
            start(b, b)

        def accum(buf, acc):
            def step(l, a):
                new = []
                for j in range(_D // 16):
                    x = buf[l, pl.ds(16 * j, 16)]
                    new.append(a[j] + x * x)
                return tuple(new)
            return lax.fori_loop(0, _CHUNK, step, acc, unroll=4)

        zeros = tuple(jnp.zeros((16,), jnp.float32) for _ in range(_D // 16))

        def group(gi, carry):
            g = gi * _NBUF
            for b in range(0, _NBUF, _CPR):
                acc = zeros
                for h in range(_CPR):
                    c = g + b + h
                    wait(c, b + h)
                    acc = accum(bufs[b + h], acc)

                    @pl.when(c + _NBUF < _NCHUNK)
                    def _():
                        start(c + _NBUF, b + h)

                row = gi * (_NBUF // _CPR) + b // _CPR
                for j in range(_D // 16):
                    out_v[row, pl.ds(16 * j, 16)] = acc[j]
            return carry

        lax.fori_loop(0, _NCHUNK // _NBUF, group, 0)
        pltpu.sync_copy(out_v, out_hbm.at[pl.ds(wid * _ROWS_W, _ROWS_W)])

    return k(ids2, table)


def _dense_tc(z3, w):
    def body(x_ref, w_ref, o_ref):
        o_ref[...] = jnp.dot(x_ref[...], w_ref[...],
                             preferred_element_type=jnp.float32)

    return pl.pallas_call(
        body,
        grid=(4,),
        in_specs=[
            pl.BlockSpec((_B // 4, _D), lambda i: (i, 0)),
            pl.BlockSpec((_D, _D), lambda i: (0, 0)),
        ],
        out_specs=pl.BlockSpec((_B // 4, _D), lambda i: (i, 0)),
        out_shape=jax.ShapeDtypeStruct((_B, _D), jnp.float32),
    )(z3, w)


def kernel(input_ids, attention_mask, embedding_table, dense_kernel):
    del attention_mask
    ids2 = input_ids.astype(jnp.int32).reshape(_B * _CPR, _CHUNK)
    z3 = _sumsq_sc(ids2, embedding_table)
    return _dense_tc(z3, dense_kernel)
